# Initial kernel scaffold; baseline (speedup 1.0000x reference)
import jax
import jax.numpy as jnp
from jax.experimental import pallas as pl


def kernel(value_ids, table, ln_weight, ln_bias):
    raise NotImplementedError("write your pallas kernel here")


# speedup vs baseline: 8.2999x; 8.2999x over previous
import functools

import jax
import jax.numpy as jnp
from jax.experimental import pallas as pl
from jax.experimental.pallas import tpu as pltpu
from jax.experimental.pallas import tpu_sc as plsc

_EPS = 1e-5
_HIDDEN = 128
_ROW_BLOCK = 256
_GATHER_W = 128


def _ln_body(x_ref, w_ref, b_ref, o_ref):
    x = x_ref[...]
    mean = jnp.mean(x, axis=1, keepdims=True)
    xc = x - mean
    var = jnp.mean(xc * xc, axis=1, keepdims=True)
    o_ref[...] = xc * jax.lax.rsqrt(var + _EPS) * w_ref[...] + b_ref[...]


def _normalize_table(table, ln_weight, ln_bias):
    rows = table.shape[0]
    grid = (rows // _ROW_BLOCK,)
    return pl.pallas_call(
        _ln_body,
        grid=grid,
        in_specs=[
            pl.BlockSpec((_ROW_BLOCK, _HIDDEN), lambda i: (i, 0)),
            pl.BlockSpec((1, _HIDDEN), lambda i: (0, 0)),
            pl.BlockSpec((1, _HIDDEN), lambda i: (0, 0)),
        ],
        out_specs=pl.BlockSpec((_ROW_BLOCK, _HIDDEN), lambda i: (i, 0)),
        out_shape=jax.ShapeDtypeStruct((rows, _HIDDEN), jnp.float32),
    )(table, ln_weight.reshape(1, _HIDDEN), ln_bias.reshape(1, _HIDDEN))


def _sc_gather(table_norm, ids_flat):
    n = ids_flat.shape[0]
    ids2 = ids_flat.reshape(1, n)
    mesh = plsc.VectorSubcoreMesh(core_axis_name="core", subcore_axis_name="subcore")

    @functools.partial(
        pl.kernel,
        out_type=jax.ShapeDtypeStruct((n, _HIDDEN), jnp.float32),
        mesh=mesh,
    )
    def k(tab_hbm, i_hbm, o_hbm):
        def body(i_vmem, o_vmem):
            pltpu.sync_copy(tab_hbm.at[i_vmem.at[0]], o_vmem)

        pltpu.emit_pipeline(
            body,
            grid=(n // _GATHER_W,),
            in_specs=[pl.BlockSpec((1, _GATHER_W), index_map=lambda i: (0, i))],
            out_specs=[pl.BlockSpec((_GATHER_W, _HIDDEN), index_map=lambda i: (i, 0))],
            core_axis_name=("core", "subcore"),
            dimension_semantics=(pltpu.PARALLEL,),
        )(i_hbm, o_hbm)

    return k(table_norm, ids2)


def kernel(value_ids, table, ln_weight, ln_bias):
    batch, seq = value_ids.shape
    vocab = table.shape[0]
    pad_rows = (-vocab) % _ROW_BLOCK
    table_p = jnp.pad(table, ((0, pad_rows), (0, 0)))
    table_norm = _normalize_table(table_p, ln_weight, ln_bias)
    out = _sc_gather(table_norm, value_ids.reshape(-1).astype(jnp.int32))
    return out.reshape(batch, seq, _HIDDEN)


# speedup vs baseline: 8.7774x; 1.0575x over previous
import functools

import jax
import jax.numpy as jnp
from jax.experimental import pallas as pl
from jax.experimental.pallas import tpu as pltpu
from jax.experimental.pallas import tpu_sc as plsc

_EPS = 1e-5
_HIDDEN = 128
_ROW_BLOCK = 256
_GATHER_W = 128
_GATHER_PER_STEP = 2


def _ln_body(x_ref, w_ref, b_ref, o_ref):
    x = x_ref[...]
    mean = jnp.mean(x, axis=1, keepdims=True)
    xc = x - mean
    var = jnp.mean(xc * xc, axis=1, keepdims=True)
    o_ref[...] = xc * jax.lax.rsqrt(var + _EPS) * w_ref[...] + b_ref[...]


def _normalize_table(table, ln_weight, ln_bias):
    rows = table.shape[0]
    grid = (rows // _ROW_BLOCK,)
    return pl.pallas_call(
        _ln_body,
        grid=grid,
        in_specs=[
            pl.BlockSpec((_ROW_BLOCK, _HIDDEN), lambda i: (i, 0)),
            pl.BlockSpec((1, _HIDDEN), lambda i: (0, 0)),
            pl.BlockSpec((1, _HIDDEN), lambda i: (0, 0)),
        ],
        out_specs=pl.BlockSpec((_ROW_BLOCK, _HIDDEN), lambda i: (i, 0)),
        out_shape=jax.ShapeDtypeStruct((rows, _HIDDEN), jnp.float32),
    )(table, ln_weight.reshape(1, _HIDDEN), ln_bias.reshape(1, _HIDDEN))


def _sc_gather(table_norm, ids_flat):
    n = ids_flat.shape[0]
    ids2 = ids_flat.reshape(1, n)
    mesh = plsc.VectorSubcoreMesh(core_axis_name="core", subcore_axis_name="subcore")

    block = _GATHER_PER_STEP * _GATHER_W

    @functools.partial(
        pl.kernel,
        out_type=jax.ShapeDtypeStruct((n, _HIDDEN), jnp.float32),
        mesh=mesh,
    )
    def k(tab_hbm, i_hbm, o_hbm):
        def body(i_vmem, o_vmem):
            for j in range(_GATHER_PER_STEP):
                pltpu.sync_copy(
                    tab_hbm.at[i_vmem.at[0, pl.ds(j * _GATHER_W, _GATHER_W)]],
                    o_vmem.at[pl.ds(j * _GATHER_W, _GATHER_W)],
                )

        pltpu.emit_pipeline(
            body,
            grid=(n // block,),
            in_specs=[pl.BlockSpec((1, block), index_map=lambda i: (0, i))],
            out_specs=[pl.BlockSpec((block, _HIDDEN), index_map=lambda i: (i, 0))],
            core_axis_name=("core", "subcore"),
            dimension_semantics=(pltpu.PARALLEL,),
        )(i_hbm, o_hbm)

    return k(table_norm, ids2)


def kernel(value_ids, table, ln_weight, ln_bias):
    batch, seq = value_ids.shape
    vocab = table.shape[0]
    pad_rows = (-vocab) % _ROW_BLOCK
    table_p = jnp.pad(table, ((0, pad_rows), (0, 0)))
    table_norm = _normalize_table(table_p, ln_weight, ln_bias)
    out = _sc_gather(table_norm, value_ids.reshape(-1).astype(jnp.int32))
    return out.reshape(batch, seq, _HIDDEN)


# speedup vs baseline: 10.2238x; 1.1648x over previous
import functools

import jax
import jax.numpy as jnp
from jax.experimental import pallas as pl
from jax.experimental.pallas import tpu as pltpu
from jax.experimental.pallas import tpu_sc as plsc

_EPS = 1e-5
_HIDDEN = 128
_ROW_BLOCK = 256
_GATHER_W = 128
_GATHER_PER_STEP = 2


def _ln_body(x_ref, w_ref, b_ref, o_ref):
    x = x_ref[...]
    mean = jnp.mean(x, axis=1, keepdims=True)
    xc = x - mean
    var = jnp.mean(xc * xc, axis=1, keepdims=True)
    o_ref[...] = xc * jax.lax.rsqrt(var + _EPS) * w_ref[...] + b_ref[...]


def _normalize_table(table, ln_weight, ln_bias):
    rows = table.shape[0]
    grid = (rows // _ROW_BLOCK,)
    return pl.pallas_call(
        _ln_body,
        grid=grid,
        in_specs=[
            pl.BlockSpec((_ROW_BLOCK, _HIDDEN), lambda i: (i, 0)),
            pl.BlockSpec((1, _HIDDEN), lambda i: (0, 0)),
            pl.BlockSpec((1, _HIDDEN), lambda i: (0, 0)),
        ],
        out_specs=pl.BlockSpec((_ROW_BLOCK, _HIDDEN), lambda i: (i, 0)),
        out_shape=jax.ShapeDtypeStruct((rows, _HIDDEN), jnp.float32),
    )(table, ln_weight.reshape(1, _HIDDEN), ln_bias.reshape(1, _HIDDEN))


_NBUF = 4


def _sc_gather(table_norm, ids_flat):
    n = ids_flat.shape[0]
    mesh = plsc.VectorSubcoreMesh(core_axis_name="core", subcore_axis_name="subcore")
    n_workers = 32
    per_w = n // n_workers
    nsteps = per_w // _GATHER_W

    @functools.partial(
        pl.kernel,
        out_type=jax.ShapeDtypeStruct((n, _HIDDEN), jnp.float32),
        mesh=mesh,
        scratch_types=[
            pltpu.VMEM((per_w,), jnp.int32),
            pltpu.VMEM((_NBUF, _GATHER_W, _HIDDEN), jnp.float32),
            pltpu.SemaphoreType.DMA((_NBUF,)),
            pltpu.SemaphoreType.DMA((_NBUF,)),
        ],
    )
    def k(tab_hbm, i_hbm, o_hbm, idx_v, rows_v, gsem, wsem):
        wid = jax.lax.axis_index("subcore") * 2 + jax.lax.axis_index("core")
        base = wid * per_w

        def gather_start(b, slot):
            pltpu.async_copy(
                tab_hbm.at[idx_v.at[pl.ds(slot * _GATHER_W, _GATHER_W)]],
                rows_v.at[b],
                gsem.at[b],
            )

        def gather_drain(b):
            pltpu.make_async_copy(
                tab_hbm.at[pl.ds(0, _GATHER_W)], rows_v.at[b], gsem.at[b]
            ).wait()

        def write_start(b, slot):
            pltpu.async_copy(
                rows_v.at[b],
                o_hbm.at[pl.ds(base + slot * _GATHER_W, _GATHER_W)],
                wsem.at[b],
            )

        def write_drain(b):
            pltpu.make_async_copy(
                rows_v.at[b], o_hbm.at[pl.ds(base, _GATHER_W)], wsem.at[b]
            ).wait()

        pltpu.sync_copy(i_hbm.at[pl.ds(base, per_w)], idx_v)

        gather_start(0, 0)
        gather_start(1, 1)
        gather_drain(0)
        write_start(0, 0)
        gather_start(2, 2)
        gather_drain(1)
        write_start(1, 1)
        gather_start(3, 3)

        @pl.loop(2, nsteps - 2, step=_NBUF)
        def _(s):
            for o in range(_NBUF):
                b = (2 + o) % _NBUF
                slot = s + o
                gather_drain(b)
                write_start(b, slot)
                nb = (b + 2) % _NBUF
                write_drain(nb)
                gather_start(nb, slot + 2)

        gather_drain((nsteps - 2) % _NBUF)
        write_start((nsteps - 2) % _NBUF, nsteps - 2)
        gather_drain((nsteps - 1) % _NBUF)
        write_start((nsteps - 1) % _NBUF, nsteps - 1)
        for b in range(_NBUF):
            write_drain(b)

    return k(table_norm, ids_flat)


def kernel(value_ids, table, ln_weight, ln_bias):
    batch, seq = value_ids.shape
    vocab = table.shape[0]
    pad_rows = (-vocab) % _ROW_BLOCK
    table_p = jnp.pad(table, ((0, pad_rows), (0, 0)))
    table_norm = _normalize_table(table_p, ln_weight, ln_bias)
    out = _sc_gather(table_norm, value_ids.reshape(-1).astype(jnp.int32))
    return out.reshape(batch, seq, _HIDDEN)


# speedup vs baseline: 10.4789x; 1.0249x over previous
import functools

import jax
import jax.numpy as jnp
from jax.experimental import pallas as pl
from jax.experimental.pallas import tpu as pltpu
from jax.experimental.pallas import tpu_sc as plsc

_EPS = 1e-5
_HIDDEN = 128
_ROW_BLOCK = 256
_GATHER_W = 128
_GATHER_PER_STEP = 2


def _ln_body(x_ref, w_ref, b_ref, o_ref):
    x = x_ref[...]
    mean = jnp.mean(x, axis=1, keepdims=True)
    xc = x - mean
    var = jnp.mean(xc * xc, axis=1, keepdims=True)
    o_ref[...] = xc * jax.lax.rsqrt(var + _EPS) * w_ref[...] + b_ref[...]


def _normalize_table(table, ln_weight, ln_bias):
    rows = table.shape[0]
    grid = (pl.cdiv(rows, _ROW_BLOCK),)
    return pl.pallas_call(
        _ln_body,
        grid=grid,
        in_specs=[
            pl.BlockSpec((_ROW_BLOCK, _HIDDEN), lambda i: (i, 0)),
            pl.BlockSpec((1, _HIDDEN), lambda i: (0, 0)),
            pl.BlockSpec((1, _HIDDEN), lambda i: (0, 0)),
        ],
        out_specs=pl.BlockSpec((_ROW_BLOCK, _HIDDEN), lambda i: (i, 0)),
        out_shape=jax.ShapeDtypeStruct((rows, _HIDDEN), jnp.float32),
    )(table, ln_weight.reshape(1, _HIDDEN), ln_bias.reshape(1, _HIDDEN))


_NBUF = 5
_PF = 3


def _sc_gather(table_norm, ids_flat):
    n = ids_flat.shape[0]
    mesh = plsc.VectorSubcoreMesh(core_axis_name="core", subcore_axis_name="subcore")
    n_workers = 32
    per_w = n // n_workers
    nsteps = per_w // _GATHER_W

    @functools.partial(
        pl.kernel,
        out_type=jax.ShapeDtypeStruct((n, _HIDDEN), jnp.float32),
        mesh=mesh,
        scratch_types=[
            pltpu.VMEM((per_w,), jnp.int32),
            pltpu.VMEM((_NBUF, _GATHER_W, _HIDDEN), jnp.float32),
            pltpu.SemaphoreType.DMA((_NBUF,)),
            pltpu.SemaphoreType.DMA((_NBUF,)),
        ],
    )
    def k(tab_hbm, i_hbm, o_hbm, idx_v, rows_v, gsem, wsem):
        wid = jax.lax.axis_index("subcore") * 2 + jax.lax.axis_index("core")
        base = wid * per_w

        def gather_start(b, slot):
            pltpu.async_copy(
                tab_hbm.at[idx_v.at[pl.ds(slot * _GATHER_W, _GATHER_W)]],
                rows_v.at[b],
                gsem.at[b],
            )

        def gather_drain(b):
            pltpu.make_async_copy(
                tab_hbm.at[pl.ds(0, _GATHER_W)], rows_v.at[b], gsem.at[b]
            ).wait()

        def write_start(b, slot):
            pltpu.async_copy(
                rows_v.at[b],
                o_hbm.at[pl.ds(base + slot * _GATHER_W, _GATHER_W)],
                wsem.at[b],
            )

        def write_drain(b):
            pltpu.make_async_copy(
                rows_v.at[b], o_hbm.at[pl.ds(base, _GATHER_W)], wsem.at[b]
            ).wait()

        pltpu.sync_copy(i_hbm.at[pl.ds(base, per_w)], idx_v)

        for s in range(_PF):
            gather_start(s, s)
        for s in range(2):
            gather_drain(s)
            write_start(s, s)
            gather_start((s + _PF) % _NBUF, s + _PF)

        @pl.loop(2, 2 + ((nsteps - _PF - 2) // _NBUF) * _NBUF, step=_NBUF)
        def _(s):
            for o in range(_NBUF):
                b = (2 + o) % _NBUF
                slot = s + o
                gather_drain(b)
                write_start(b, slot)
                nb = (b + _PF) % _NBUF
                write_drain(nb)
                gather_start(nb, slot + _PF)

        tail_start = 2 + ((nsteps - _PF - 2) // _NBUF) * _NBUF
        for slot in range(tail_start, nsteps):
            b = slot % _NBUF
            gather_drain(b)
            write_start(b, slot)
            if slot + _PF < nsteps:
                nb = (b + _PF) % _NBUF
                write_drain(nb)
                gather_start(nb, slot + _PF)
        for slot in range(nsteps - _NBUF, nsteps):
            write_drain(slot % _NBUF)

    return k(table_norm, ids_flat)


def kernel(value_ids, table, ln_weight, ln_bias):
    batch, seq = value_ids.shape
    table_norm = _normalize_table(table, ln_weight, ln_bias)
    out = _sc_gather(table_norm, value_ids.reshape(-1).astype(jnp.int32))
    return out.reshape(batch, seq, _HIDDEN)
